# baseline (speedup 1.0000x reference)
import functools

import jax
import jax.numpy as jnp
from jax import lax
from jax.experimental import pallas as pl
from jax.experimental.pallas import tpu as pltpu
from jax.experimental.pallas import tpu_sc as plsc

NU, NI, D = 4000, 6000, 64
N = NU + NI
NPAD = 10240
E = 320000
NC, NS = 2, 16
NH = 5120
NHALF = 5000
LTRASH = 5000
CHUNK = 128
EPT = 10752
NITER = EPT // CHUNK
CAP = NS * EPT
RPT = NPAD // NS
APT = NH // NS

NBUF = 4
LEAD = 2
DW = 16
TSLOT = N * N
TK = N * N + CHUNK

_mesh = plsc.VectorSubcoreMesh(core_axis_name="c", subcore_axis_name="s")


@functools.partial(
    pl.kernel,
    out_type=jax.ShapeDtypeStruct((NC, NH, D), jnp.float32),
    mesh=_mesh,
    scratch_types=[
        pltpu.VMEM((NITER, CHUNK), jnp.int32),
        pltpu.VMEM((NITER, CHUNK), jnp.int32),
        [pltpu.VMEM((CHUNK, D), jnp.float32)] * NBUF,
        [pltpu.SemaphoreType.DMA] * NBUF,
        [pltpu.SemaphoreType.DMA] * NBUF,
        pltpu.VMEM_SHARED((NH, D), jnp.float32),
        pltpu.VMEM_SHARED((NPAD, D), jnp.float32),
    ],
    compiler_params=pltpu.CompilerParams(use_tc_tiling_on_sc=False),
)
def _sc_segsum(z_hbm, cols_hbm, rows_hbm, out_hbm,
               colv, rowv, gbufs, gsems, ssems, acc_sh, z_sh):
    cid = lax.axis_index("c")
    sid = lax.axis_index("s")
    blk = cid * NS + sid

    zr = pl.multiple_of(sid * RPT, 8)
    pltpu.sync_copy(z_hbm.at[pl.ds(zr, RPT)], z_sh.at[pl.ds(zr, RPT)])
    ar = pl.multiple_of(sid * APT, 8)
    zsrc = pl.multiple_of(cid * NHALF + sid * APT, 8)
    pltpu.sync_copy(z_hbm.at[pl.ds(zsrc, APT)], acc_sh.at[pl.ds(ar, APT)])
    pltpu.sync_copy(cols_hbm.at[blk], colv)
    pltpu.sync_copy(rows_hbm.at[blk], rowv)
    plsc.subcore_barrier()

    def gather(i, b):
        pltpu.async_copy(z_sh.at[colv.at[i]], gbufs[b], gsems[b])

    def gather_wait(i, b):
        pltpu.make_async_copy(z_sh.at[colv.at[i]], gbufs[b], gsems[b]).wait()

    def scat(i, b):
        pltpu.async_copy(gbufs[b], acc_sh.at[rowv.at[i]], ssems[b], add=True)

    def scat_wait(i, b):
        pltpu.make_async_copy(gbufs[b], acc_sh.at[rowv.at[i]],
                              ssems[b]).wait()

    for b in range(LEAD):
        gather(b, b)

    def step(j, c):
        for b in range(NBUF):
            i = j * NBUF + b
            gather_wait(i, b)
            scat(i, b)
            bn = (b + LEAD) % NBUF

            @pl.when(i - (NBUF - LEAD) >= 0)
            def _():
                scat_wait(i - (NBUF - LEAD), bn)

            @pl.when(i + LEAD < NITER)
            def _():
                gather(i + LEAD, bn)
        return c

    lax.fori_loop(0, NITER // NBUF, step, 0)
    for k in range(NITER - (NBUF - LEAD), NITER):
        scat_wait(k, k % NBUF)
    plsc.subcore_barrier()

    @pl.when(sid == 0)
    def _():
        pltpu.sync_copy(acc_sh, out_hbm.at[cid])


@functools.partial(
    pl.kernel,
    out_type=[
        jax.ShapeDtypeStruct((NC * NS, NITER, CHUNK), jnp.int32),
        jax.ShapeDtypeStruct((NC, NH, DW), jnp.float32),
        jax.ShapeDtypeStruct((TK,), jnp.int32),
    ],
    mesh=_mesh,
    scratch_types=[
        pltpu.VMEM((NITER, CHUNK), jnp.int32),
        pltpu.VMEM((NITER, CHUNK), jnp.int32),
        pltpu.VMEM((NITER, CHUNK), jnp.int32),
        pltpu.VMEM((NITER, CHUNK), jnp.int32),
        [pltpu.VMEM((CHUNK,), jnp.int32)] * NBUF,
        [pltpu.VMEM((CHUNK,), jnp.int32)] * NBUF,
        pltpu.VMEM((CHUNK, DW), jnp.float32),
        pltpu.VMEM((NH // NS, DW), jnp.float32),
        [pltpu.SemaphoreType.DMA] * NBUF,
        [pltpu.SemaphoreType.DMA] * NBUF,
        [pltpu.SemaphoreType.DMA] * NBUF,
        pltpu.VMEM_SHARED((NH, DW), jnp.float32),
    ],
    compiler_params=pltpu.CompilerParams(use_tc_tiling_on_sc=False),
)
def _sc_dedup(k_hbm, r_hbm, i_hbm, lrow_hbm, deg_hbm, t_hbm,
              kv, rv, iv, rowout, kwbufs, abufs, onesb, zbuf,
              wsems, gsems, dsems, deg_sh):
    cid = lax.axis_index("c")
    sid = lax.axis_index("s")
    blk = cid * NS + sid
    lo = cid * NHALF

    pltpu.sync_copy(k_hbm.at[blk], kv)
    pltpu.sync_copy(r_hbm.at[blk], rv)
    pltpu.sync_copy(i_hbm.at[blk], iv)
    for r in range(CHUNK):
        onesb[r, :] = jnp.ones((DW,), jnp.float32)
    for r in range(NH // NS):
        zbuf[r, :] = jnp.zeros((DW,), jnp.float32)
    zr = pl.multiple_of(sid * (NH // NS), 8)
    pltpu.sync_copy(zbuf, deg_sh.at[pl.ds(zr, NH // NS)])
    plsc.subcore_barrier()

    def compute_kw(i, b):
        for jj in range(CHUNK // 16):
            sl = pl.ds(jj * 16, 16)
            kk = kv[i, sl]
            rr = rv[i, sl]
            ow = (rr >= lo) & (rr < lo + NHALF)
            kwbufs[b][sl] = jnp.where(ow, kk, TSLOT)

    def wscat(i, b):
        pltpu.async_copy(iv.at[i], t_hbm.at[kwbufs[b]], wsems[b])

    def wscat_wait(i, b):
        pltpu.make_async_copy(iv.at[i], t_hbm.at[kwbufs[b]],
                              wsems[b]).wait()

    def stepA(j, c):
        for b in range(NBUF):
            i = j * NBUF + b

            @pl.when(i - NBUF >= 0)
            def _():
                wscat_wait(i - NBUF, b)

            compute_kw(i, b)
            wscat(i, b)
        return c

    lax.fori_loop(0, NITER // NBUF, stepA, 0)
    for k in range(NITER - NBUF, NITER):
        wscat_wait(k, k % NBUF)
    plsc.subcore_barrier()

    def gather(i, b):
        pltpu.async_copy(t_hbm.at[kwbufs[b]], abufs[b], gsems[b])

    def gather_wait(i, b):
        pltpu.make_async_copy(t_hbm.at[kwbufs[b]], abufs[b],
                              gsems[b]).wait()

    def dscat(i, b):
        pltpu.async_copy(onesb, deg_sh.at[rowout.at[i]], dsems[b], add=True)

    def dscat_wait(i, b):
        pltpu.make_async_copy(onesb, deg_sh.at[rowout.at[i]],
                              dsems[b]).wait()

    def emit(i, b):
        for jj in range(CHUNK // 16):
            sl = pl.ds(jj * 16, 16)
            aa = abufs[b][sl]
            ii = iv[i, sl]
            rr = rv[i, sl]
            ow = (rr >= lo) & (rr < lo + NHALF)
            keep = (aa == ii) & ow
            rowout[i, sl] = jnp.where(keep, rr - lo, LTRASH)

    for b in range(LEAD):
        compute_kw(b, b)
        gather(b, b)

    def stepB(j, c):
        for b in range(NBUF):
            i = j * NBUF + b
            gather_wait(i, b)
            emit(i, b)
            dscat(i, b)
            bn = (b + LEAD) % NBUF

            @pl.when(i - (NBUF - LEAD) >= 0)
            def _():
                dscat_wait(i - (NBUF - LEAD), bn)

            @pl.when(i + LEAD < NITER)
            def _():
                compute_kw(i + LEAD, bn)
                gather(i + LEAD, bn)
        return c

    lax.fori_loop(0, NITER // NBUF, stepB, 0)
    for k in range(NITER - (NBUF - LEAD), NITER):
        dscat_wait(k, k % NBUF)
    pltpu.sync_copy(rowout, lrow_hbm.at[blk])
    plsc.subcore_barrier()

    @pl.when(sid == 0)
    def _():
        pltpu.sync_copy(deg_sh, deg_hbm.at[cid])


def _prep_body(degp_ref, u_ref, it_ref, z0_ref, s2b_ref):
    deg = jnp.concatenate([degp_ref[0, 0:NHALF, 0:1],
                           degp_ref[1, 0:NHALF, 0:1]], axis=0) + 1.0
    s = lax.rsqrt(deg)
    s2b_ref[pl.ds(0, N)] = jnp.broadcast_to(1.0 / deg, (N, D))
    s2b_ref[pl.ds(N, NPAD - N)] = jnp.ones((NPAD - N, D), jnp.float32)
    z0_ref[pl.ds(0, NU)] = s[0:NU] * u_ref[...]
    z0_ref[pl.ds(NU, NI)] = s[NU:N] * it_ref[...]
    z0_ref[pl.ds(N, NPAD - N)] = jnp.zeros((NPAD - N, D), jnp.float32)


def _acc_body(p_ref, s2b_ref, t_ref, zn_ref, tn_ref):
    az = jnp.concatenate([p_ref[0, 0:NHALF], p_ref[1, 0:NHALF]], axis=0)
    zn = s2b_ref[0:N] * az
    zn_ref[pl.ds(0, N)] = zn
    zn_ref[pl.ds(N, NPAD - N)] = jnp.zeros((NPAD - N, D), jnp.float32)
    tn_ref[pl.ds(0, N)] = t_ref[0:N] + zn
    tn_ref[pl.ds(N, NPAD - N)] = jnp.zeros((NPAD - N, D), jnp.float32)


def _final_body(p_ref, s2b_ref, t_ref, u_ref, it_ref):
    az = jnp.concatenate([p_ref[0, 0:NHALF], p_ref[1, 0:NHALF]], axis=0)
    s2 = s2b_ref[0:N]
    out = (t_ref[0:N] + s2 * az) * lax.rsqrt(s2)
    u_ref[...] = out[0:NU]
    it_ref[...] = out[NU:N]


_prep = pl.pallas_call(
    _prep_body,
    out_shape=[jax.ShapeDtypeStruct((NPAD, D), jnp.float32),
               jax.ShapeDtypeStruct((NPAD, D), jnp.float32)],
)

_acc = pl.pallas_call(
    _acc_body,
    out_shape=[jax.ShapeDtypeStruct((NPAD, D), jnp.float32),
               jax.ShapeDtypeStruct((NPAD, D), jnp.float32)],
)

_final = pl.pallas_call(
    _final_body,
    out_shape=[jax.ShapeDtypeStruct((NU, D), jnp.float32),
               jax.ShapeDtypeStruct((NI, D), jnp.float32)],
)


def kernel(user_table, item_table, edge_index):
    e0 = edge_index[0]
    e1 = edge_index[1]
    keys = e0 * N + e1

    idx = jnp.arange(E, dtype=jnp.int32)
    side0 = e0 < NHALF
    cs0 = jnp.cumsum(side0.astype(jnp.int32))
    c0 = cs0[E - 1]
    pos = jnp.where(side0, cs0 - 1, c0 + idx - cs0)
    zed = jnp.zeros((E,), jnp.int32)
    kperm = zed.at[pos].set(keys, unique_indices=True)
    rperm = zed.at[pos].set(e0, unique_indices=True)
    cperm = zed.at[pos].set(e1, unique_indices=True)

    def blocks(a):
        return jnp.concatenate([a[0:CAP], a[E - CAP:E]]).reshape(
            NC * NS, NITER, CHUNK)

    rows_p, degp, _ = _sc_dedup(blocks(kperm), blocks(rperm), blocks(idx))
    cols_p = blocks(cperm)

    z0, s2b = _prep(degp, user_table, item_table)
    p = _sc_segsum(z0, cols_p, rows_p)
    z1, t1 = _acc(p, s2b, z0)
    p = _sc_segsum(z1, cols_p, rows_p)
    z2, t2 = _acc(p, s2b, t1)
    p = _sc_segsum(z2, cols_p, rows_p)
    return _final(p, s2b, t2)


# speedup vs baseline: 9.1092x; 9.1092x over previous
import functools

import jax
import jax.numpy as jnp
from jax import lax
from jax.experimental import pallas as pl
from jax.experimental.pallas import tpu as pltpu
from jax.experimental.pallas import tpu_sc as plsc

NU, NI, D = 4000, 6000, 64
N = NU + NI
NPAD = 10240
E = 320000
NC, NS = 2, 16
NH = 5120
NHALF = 5000
LTRASH = 5000
CHUNK = 128
EPT = 10752
NITER = EPT // CHUNK
CAP = NS * EPT
RPT = NPAD // NS
APT = NH // NS

NBUF = 4
LEAD = 2
DW = 16

_mesh = plsc.VectorSubcoreMesh(core_axis_name="c", subcore_axis_name="s")


@functools.partial(
    pl.kernel,
    out_type=jax.ShapeDtypeStruct((NC, NH, D), jnp.float32),
    mesh=_mesh,
    scratch_types=[
        pltpu.VMEM((NITER, CHUNK), jnp.int32),
        pltpu.VMEM((NITER, CHUNK), jnp.int32),
        [pltpu.VMEM((CHUNK, D), jnp.float32)] * NBUF,
        [pltpu.SemaphoreType.DMA] * NBUF,
        [pltpu.SemaphoreType.DMA] * NBUF,
        pltpu.VMEM_SHARED((NH, D), jnp.float32),
        pltpu.VMEM_SHARED((NPAD, D), jnp.float32),
    ],
    compiler_params=pltpu.CompilerParams(use_tc_tiling_on_sc=False),
)
def _sc_segsum(z_hbm, cols_hbm, rows_hbm, out_hbm,
               colv, rowv, gbufs, gsems, ssems, acc_sh, z_sh):
    cid = lax.axis_index("c")
    sid = lax.axis_index("s")
    blk = cid * NS + sid

    zr = pl.multiple_of(sid * RPT, 8)
    pltpu.sync_copy(z_hbm.at[pl.ds(zr, RPT)], z_sh.at[pl.ds(zr, RPT)])
    ar = pl.multiple_of(sid * APT, 8)
    zsrc = pl.multiple_of(cid * NHALF + sid * APT, 8)
    pltpu.sync_copy(z_hbm.at[pl.ds(zsrc, APT)], acc_sh.at[pl.ds(ar, APT)])
    pltpu.sync_copy(cols_hbm.at[blk], colv)
    pltpu.sync_copy(rows_hbm.at[blk], rowv)
    plsc.subcore_barrier()

    def gather(i, b):
        pltpu.async_copy(z_sh.at[colv.at[i]], gbufs[b], gsems[b])

    def gather_wait(i, b):
        pltpu.make_async_copy(z_sh.at[colv.at[i]], gbufs[b], gsems[b]).wait()

    def scat(i, b):
        pltpu.async_copy(gbufs[b], acc_sh.at[rowv.at[i]], ssems[b], add=True)

    def scat_wait(i, b):
        pltpu.make_async_copy(gbufs[b], acc_sh.at[rowv.at[i]],
                              ssems[b]).wait()

    for b in range(LEAD):
        gather(b, b)

    def step(j, c):
        for b in range(NBUF):
            i = j * NBUF + b
            gather_wait(i, b)
            scat(i, b)
            bn = (b + LEAD) % NBUF

            @pl.when(i - (NBUF - LEAD) >= 0)
            def _():
                scat_wait(i - (NBUF - LEAD), bn)

            @pl.when(i + LEAD < NITER)
            def _():
                gather(i + LEAD, bn)
        return c

    lax.fori_loop(0, NITER // NBUF, step, 0)
    for k in range(NITER - (NBUF - LEAD), NITER):
        scat_wait(k, k % NBUF)
    plsc.subcore_barrier()

    @pl.when(sid == 0)
    def _():
        pltpu.sync_copy(acc_sh, out_hbm.at[cid])


@functools.partial(
    pl.kernel,
    out_type=jax.ShapeDtypeStruct((NC, NH, DW), jnp.float32),
    mesh=_mesh,
    scratch_types=[
        pltpu.VMEM((NITER, CHUNK), jnp.int32),
        pltpu.VMEM((CHUNK, DW), jnp.float32),
        pltpu.VMEM((APT, DW), jnp.float32),
        [pltpu.SemaphoreType.DMA] * NBUF,
        pltpu.VMEM_SHARED((NH, DW), jnp.float32),
    ],
    compiler_params=pltpu.CompilerParams(use_tc_tiling_on_sc=False),
)
def _sc_degree(rows_hbm, out_hbm, rowv, onesb, zbuf, dsems, deg_sh):
    cid = lax.axis_index("c")
    sid = lax.axis_index("s")
    blk = cid * NS + sid

    pltpu.sync_copy(rows_hbm.at[blk], rowv)
    for r in range(CHUNK):
        onesb[r, :] = jnp.ones((DW,), jnp.float32)
    for r in range(APT):
        zbuf[r, :] = jnp.zeros((DW,), jnp.float32)
    zr = pl.multiple_of(sid * APT, 8)
    pltpu.sync_copy(zbuf, deg_sh.at[pl.ds(zr, APT)])
    plsc.subcore_barrier()

    def dscat(i, b):
        pltpu.async_copy(onesb, deg_sh.at[rowv.at[i]], dsems[b], add=True)

    def dwait(i, b):
        pltpu.make_async_copy(onesb, deg_sh.at[rowv.at[i]],
                              dsems[b]).wait()

    def step(j, c):
        for b in range(NBUF):
            i = j * NBUF + b

            @pl.when(i - NBUF >= 0)
            def _():
                dwait(i - NBUF, b)

            dscat(i, b)
        return c

    lax.fori_loop(0, NITER // NBUF, step, 0)
    for k in range(NITER - NBUF, NITER):
        dwait(k, k % NBUF)
    plsc.subcore_barrier()

    @pl.when(sid == 0)
    def _():
        pltpu.sync_copy(deg_sh, out_hbm.at[cid])


def _prep_body(degp_ref, u_ref, it_ref, z0_ref, s2b_ref):
    deg = jnp.concatenate([degp_ref[0, 0:NHALF, 0:1],
                           degp_ref[1, 0:NHALF, 0:1]], axis=0) + 1.0
    s = lax.rsqrt(deg)
    s2b_ref[pl.ds(0, N)] = jnp.broadcast_to(1.0 / deg, (N, D))
    s2b_ref[pl.ds(N, NPAD - N)] = jnp.ones((NPAD - N, D), jnp.float32)
    z0_ref[pl.ds(0, NU)] = s[0:NU] * u_ref[...]
    z0_ref[pl.ds(NU, NI)] = s[NU:N] * it_ref[...]
    z0_ref[pl.ds(N, NPAD - N)] = jnp.zeros((NPAD - N, D), jnp.float32)


def _acc_body(p_ref, s2b_ref, t_ref, zn_ref, tn_ref):
    az = jnp.concatenate([p_ref[0, 0:NHALF], p_ref[1, 0:NHALF]], axis=0)
    zn = s2b_ref[0:N] * az
    zn_ref[pl.ds(0, N)] = zn
    zn_ref[pl.ds(N, NPAD - N)] = jnp.zeros((NPAD - N, D), jnp.float32)
    tn_ref[pl.ds(0, N)] = t_ref[0:N] + zn
    tn_ref[pl.ds(N, NPAD - N)] = jnp.zeros((NPAD - N, D), jnp.float32)


def _final_body(p_ref, s2b_ref, t_ref, u_ref, it_ref):
    az = jnp.concatenate([p_ref[0, 0:NHALF], p_ref[1, 0:NHALF]], axis=0)
    s2 = s2b_ref[0:N]
    out = (t_ref[0:N] + s2 * az) * lax.rsqrt(s2)
    u_ref[...] = out[0:NU]
    it_ref[...] = out[NU:N]


_prep = pl.pallas_call(
    _prep_body,
    out_shape=[jax.ShapeDtypeStruct((NPAD, D), jnp.float32),
               jax.ShapeDtypeStruct((NPAD, D), jnp.float32)],
)

_acc = pl.pallas_call(
    _acc_body,
    out_shape=[jax.ShapeDtypeStruct((NPAD, D), jnp.float32),
               jax.ShapeDtypeStruct((NPAD, D), jnp.float32)],
)

_final = pl.pallas_call(
    _final_body,
    out_shape=[jax.ShapeDtypeStruct((NU, D), jnp.float32),
               jax.ShapeDtypeStruct((NI, D), jnp.float32)],
)


def kernel(user_table, item_table, edge_index):
    keys = edge_index[0] * N + edge_index[1]
    sk = jnp.sort(keys)
    rows = sk // N
    cols = sk % N
    first = jnp.concatenate(
        [jnp.ones((1,), jnp.bool_), sk[1:] != sk[:-1]])
    rows_eff = jnp.where(first, rows, N)

    c0 = jnp.sum((edge_index[0] < NHALF).astype(jnp.int32))
    r_blk0 = rows_eff[0:CAP]
    c_blk0 = cols[0:CAP]
    lrow0 = jnp.where(r_blk0 < NHALF, r_blk0, LTRASH)
    r_pad = jnp.concatenate([rows_eff, jnp.full((CAP,), N, jnp.int32)])
    c_pad = jnp.concatenate([cols, jnp.zeros((CAP,), jnp.int32)])
    r_blk1 = lax.dynamic_slice(r_pad, (c0,), (CAP,))
    c_blk1 = lax.dynamic_slice(c_pad, (c0,), (CAP,))
    lrow1 = jnp.where((r_blk1 >= NHALF) & (r_blk1 < N),
                      r_blk1 - NHALF, LTRASH)
    rows_p = jnp.stack([lrow0, lrow1]).reshape(NC * NS, NITER, CHUNK)
    cols_p = jnp.stack([c_blk0, c_blk1]).reshape(NC * NS, NITER, CHUNK)

    degp = _sc_degree(rows_p)
    z0, s2b = _prep(degp, user_table, item_table)
    p = _sc_segsum(z0, cols_p, rows_p)
    z1, t1 = _acc(p, s2b, z0)
    p = _sc_segsum(z1, cols_p, rows_p)
    z2, t2 = _acc(p, s2b, t1)
    p = _sc_segsum(z2, cols_p, rows_p)
    return _final(p, s2b, t2)


# speedup vs baseline: 9.4012x; 1.0320x over previous
import functools

import jax
import jax.numpy as jnp
from jax import lax
from jax.experimental import pallas as pl
from jax.experimental.pallas import tpu as pltpu
from jax.experimental.pallas import tpu_sc as plsc

NU, NI, D = 4000, 6000, 64
N = NU + NI
NPAD = 10240
E = 320000
NC, NS = 2, 16
NH = 5120
NHALF = 5000
LTRASH = 5000
CHUNK = 128
EPT = 10752
NITER = EPT // CHUNK
CAP = NS * EPT
RPT = NPAD // NS
APT = NH // NS

NBUF = 4
LEAD = 2
DW = 16

_mesh = plsc.VectorSubcoreMesh(core_axis_name="c", subcore_axis_name="s")


@functools.partial(
    pl.kernel,
    out_type=jax.ShapeDtypeStruct((NC, NH, D), jnp.float32),
    mesh=_mesh,
    scratch_types=[
        pltpu.VMEM((NITER, CHUNK), jnp.int32),
        pltpu.VMEM((NITER, CHUNK), jnp.int32),
        [pltpu.VMEM((CHUNK, D), jnp.float32)] * NBUF,
        [pltpu.SemaphoreType.DMA] * NBUF,
        [pltpu.SemaphoreType.DMA] * NBUF,
        pltpu.VMEM_SHARED((NH, D), jnp.float32),
        pltpu.VMEM_SHARED((NPAD, D), jnp.float32),
    ],
    compiler_params=pltpu.CompilerParams(use_tc_tiling_on_sc=False),
)
def _sc_segsum(z_hbm, cols_hbm, rows_hbm, out_hbm,
               colv, rowv, gbufs, gsems, ssems, acc_sh, z_sh):
    cid = lax.axis_index("c")
    sid = lax.axis_index("s")
    blk = cid * NS + sid

    zr = pl.multiple_of(sid * RPT, 8)
    pltpu.sync_copy(z_hbm.at[pl.ds(zr, RPT)], z_sh.at[pl.ds(zr, RPT)])
    ar = pl.multiple_of(sid * APT, 8)
    zsrc = pl.multiple_of(cid * NHALF + sid * APT, 8)
    pltpu.sync_copy(z_hbm.at[pl.ds(zsrc, APT)], acc_sh.at[pl.ds(ar, APT)])
    pltpu.sync_copy(cols_hbm.at[blk], colv)
    pltpu.sync_copy(rows_hbm.at[blk], rowv)
    plsc.subcore_barrier()

    def gather(i, b):
        pltpu.async_copy(z_sh.at[colv.at[i]], gbufs[b], gsems[b])

    def gather_wait(i, b):
        pltpu.make_async_copy(z_sh.at[colv.at[i]], gbufs[b], gsems[b]).wait()

    def scat(i, b):
        pltpu.async_copy(gbufs[b], acc_sh.at[rowv.at[i]], ssems[b], add=True)

    def scat_wait(i, b):
        pltpu.make_async_copy(gbufs[b], acc_sh.at[rowv.at[i]],
                              ssems[b]).wait()

    for b in range(LEAD):
        gather(b, b)

    def step(j, c):
        for b in range(NBUF):
            i = j * NBUF + b
            gather_wait(i, b)
            scat(i, b)
            bn = (b + LEAD) % NBUF

            @pl.when(i - (NBUF - LEAD) >= 0)
            def _():
                scat_wait(i - (NBUF - LEAD), bn)

            @pl.when(i + LEAD < NITER)
            def _():
                gather(i + LEAD, bn)
        return c

    lax.fori_loop(0, NITER // NBUF, step, 0)
    for k in range(NITER - (NBUF - LEAD), NITER):
        scat_wait(k, k % NBUF)
    plsc.subcore_barrier()

    @pl.when(sid == 0)
    def _():
        pltpu.sync_copy(acc_sh, out_hbm.at[cid])


@functools.partial(
    pl.kernel,
    out_type=jax.ShapeDtypeStruct((NC, NH, DW), jnp.float32),
    mesh=_mesh,
    scratch_types=[
        pltpu.VMEM((NITER, CHUNK), jnp.int32),
        pltpu.VMEM((CHUNK, DW), jnp.float32),
        pltpu.VMEM((APT, DW), jnp.float32),
        [pltpu.SemaphoreType.DMA] * NBUF,
        pltpu.VMEM_SHARED((NH, DW), jnp.float32),
    ],
    compiler_params=pltpu.CompilerParams(use_tc_tiling_on_sc=False),
)
def _sc_degree(rows_hbm, out_hbm, rowv, onesb, zbuf, dsems, deg_sh):
    cid = lax.axis_index("c")
    sid = lax.axis_index("s")
    blk = cid * NS + sid

    pltpu.sync_copy(rows_hbm.at[blk], rowv)
    for r in range(CHUNK):
        onesb[r, :] = jnp.ones((DW,), jnp.float32)
    for r in range(APT):
        zbuf[r, :] = jnp.zeros((DW,), jnp.float32)
    zr = pl.multiple_of(sid * APT, 8)
    pltpu.sync_copy(zbuf, deg_sh.at[pl.ds(zr, APT)])
    plsc.subcore_barrier()

    def dscat(i, b):
        pltpu.async_copy(onesb, deg_sh.at[rowv.at[i]], dsems[b], add=True)

    def dwait(i, b):
        pltpu.make_async_copy(onesb, deg_sh.at[rowv.at[i]],
                              dsems[b]).wait()

    def step(j, c):
        for b in range(NBUF):
            i = j * NBUF + b

            @pl.when(i - NBUF >= 0)
            def _():
                dwait(i - NBUF, b)

            dscat(i, b)
        return c

    lax.fori_loop(0, NITER // NBUF, step, 0)
    for k in range(NITER - NBUF, NITER):
        dwait(k, k % NBUF)
    plsc.subcore_barrier()

    @pl.when(sid == 0)
    def _():
        pltpu.sync_copy(deg_sh, out_hbm.at[cid])


def _prep_body(degp_ref, u_ref, it_ref, z0_ref, s2b_ref):
    deg = jnp.concatenate([degp_ref[0, 0:NHALF, 0:1],
                           degp_ref[1, 0:NHALF, 0:1]], axis=0) + 1.0
    s = lax.rsqrt(deg)
    s2b_ref[pl.ds(0, N)] = jnp.broadcast_to(1.0 / deg, (N, D))
    s2b_ref[pl.ds(N, NPAD - N)] = jnp.ones((NPAD - N, D), jnp.float32)
    z0_ref[pl.ds(0, NU)] = s[0:NU] * u_ref[...]
    z0_ref[pl.ds(NU, NI)] = s[NU:N] * it_ref[...]
    z0_ref[pl.ds(N, NPAD - N)] = jnp.zeros((NPAD - N, D), jnp.float32)


def _acc_body(p_ref, s2b_ref, t_ref, zn_ref, tn_ref):
    az = jnp.concatenate([p_ref[0, 0:NHALF], p_ref[1, 0:NHALF]], axis=0)
    zn = s2b_ref[0:N] * az
    zn_ref[pl.ds(0, N)] = zn
    zn_ref[pl.ds(N, NPAD - N)] = jnp.zeros((NPAD - N, D), jnp.float32)
    tn_ref[pl.ds(0, N)] = t_ref[0:N] + zn
    tn_ref[pl.ds(N, NPAD - N)] = jnp.zeros((NPAD - N, D), jnp.float32)


def _final_body(p_ref, s2b_ref, t_ref, u_ref, it_ref):
    az = jnp.concatenate([p_ref[0, 0:NHALF], p_ref[1, 0:NHALF]], axis=0)
    s2 = s2b_ref[0:N]
    out = (t_ref[0:N] + s2 * az) * lax.rsqrt(s2)
    u_ref[...] = out[0:NU]
    it_ref[...] = out[NU:N]


_prep = pl.pallas_call(
    _prep_body,
    out_shape=[jax.ShapeDtypeStruct((NPAD, D), jnp.float32),
               jax.ShapeDtypeStruct((NPAD, D), jnp.float32)],
)

_acc = pl.pallas_call(
    _acc_body,
    out_shape=[jax.ShapeDtypeStruct((NPAD, D), jnp.float32),
               jax.ShapeDtypeStruct((NPAD, D), jnp.float32)],
)

_final = pl.pallas_call(
    _final_body,
    out_shape=[jax.ShapeDtypeStruct((NU, D), jnp.float32),
               jax.ShapeDtypeStruct((NI, D), jnp.float32)],
)


def kernel(user_table, item_table, edge_index):
    keys = edge_index[0] * N + edge_index[1]
    sk = jnp.sort(keys)
    rows = sk // N
    cols = sk % N
    first = jnp.concatenate(
        [jnp.ones((1,), jnp.bool_), sk[1:] != sk[:-1]])
    rows_eff = jnp.where(first, rows, N)

    r_blk0 = rows_eff[0:CAP]
    c_blk0 = cols[0:CAP]
    lrow0 = jnp.where(r_blk0 < NHALF, r_blk0, LTRASH)
    r_blk1 = rows_eff[E - CAP:E]
    c_blk1 = cols[E - CAP:E]
    lrow1 = jnp.where((r_blk1 >= NHALF) & (r_blk1 < N),
                      r_blk1 - NHALF, LTRASH)
    rows_p = jnp.stack([lrow0, lrow1]).reshape(NC * NS, NITER, CHUNK)
    cols_p = jnp.stack([c_blk0, c_blk1]).reshape(NC * NS, NITER, CHUNK)

    degp = _sc_degree(rows_p)
    z0, s2b = _prep(degp, user_table, item_table)
    p = _sc_segsum(z0, cols_p, rows_p)
    z1, t1 = _acc(p, s2b, z0)
    p = _sc_segsum(z1, cols_p, rows_p)
    z2, t2 = _acc(p, s2b, t1)
    p = _sc_segsum(z2, cols_p, rows_p)
    return _final(p, s2b, t2)
